# baseline (speedup 1.0000x reference)
import functools

import jax
import jax.numpy as jnp
from jax import lax
from jax.experimental import pallas as pl
from jax.experimental.pallas import tpu as pltpu
from jax.experimental.pallas import tpu_sc as plsc

NUM_WORKERS = 32
CB = 8
NBUF = 4


def _repack_table(table, v, d):
    tt = table.T
    n_full = v // 128
    mesh = plsc.VectorSubcoreMesh(core_axis_name="c", subcore_axis_name="s")

    @functools.partial(
        pl.kernel,
        mesh=mesh,
        out_type=jax.ShapeDtypeStruct((v // 2, 2 * d), jnp.float32),
        scratch_types=[
            [pltpu.VMEM((d, 128), jnp.float32) for _ in range(2)],
            [pltpu.VMEM((d, 128), jnp.float32) for _ in range(2)],
            [pltpu.SemaphoreType.DMA for _ in range(2)],
            [pltpu.SemaphoreType.DMA for _ in range(2)],
        ],
        compiler_params=pltpu.CompilerParams(
            use_tc_tiling_on_sc=True, needs_layout_passes=False
        ),
    )
    def t(tt_hbm, out_hbm, bufs, tbufs, semi, semo):
        wid = lax.axis_index("s") * 2 + lax.axis_index("c")
        lanes = lax.iota(jnp.int32, 16)

        def blk(k):
            return wid + NUM_WORKERS * k

        def start_in(k, p):
            pltpu.async_copy(
                tt_hbm.at[:, pl.ds(blk(k) * 128, 128)], bufs[p], semi[p]
            )

        def wait_in(k, p):
            pltpu.make_async_copy(
                tt_hbm.at[:, pl.ds(blk(k) * 128, 128)], bufs[p], semi[p]
            ).wait()

        def start_out(k, p):
            pltpu.async_copy(
                tbufs[p], out_hbm.at[pl.ds(blk(k) * 64, 64)], semo[p]
            )

        def wait_out_any(p):
            pltpu.make_async_copy(
                tbufs[p], out_hbm.at[pl.ds(0, 64)], semo[p]
            ).wait()

        def transpose(p, width):
            @plsc.parallel_loop(0, width, 1, unroll=16)
            def _(i):
                ivec = jnp.zeros((16,), jnp.int32) + i
                row = i // 2
                col = (i % 2) * d
                for c in range(d // 16):
                    vals = plsc.load_gather(bufs[p], [lanes + 16 * c, ivec])
                    tbufs[p][row, pl.ds(col + 16 * c, 16)] = vals

        nb = (n_full - 1 - wid) // NUM_WORKERS + 1
        npairs = nb // 2

        start_in(0, 0)

        def body(g, carry):
            k0 = 2 * g
            k1 = 2 * g + 1

            wait_in(k0, 0)
            start_in(k1, 1)

            @pl.when(k0 >= 2)
            def _():
                wait_out_any(0)

            transpose(0, 128)
            start_out(k0, 0)

            wait_in(k1, 1)

            @pl.when(k1 + 1 < nb)
            def _():
                start_in(k1 + 1, 0)

            @pl.when(k1 >= 2)
            def _():
                wait_out_any(1)

            transpose(1, 128)
            start_out(k1, 1)
            return carry

        lax.fori_loop(0, npairs, body, 0)

        @pl.when(2 * npairs < nb)
        def _():
            k = nb - 1
            wait_in(k, 0)
            wait_out_any(0)
            transpose(0, 128)
            start_out(k, 0)

        wait_out_any(0)
        wait_out_any(1)

        @pl.when(wid == (n_full % NUM_WORKERS))
        def _():
            tail_start = jnp.asarray(n_full * 128, jnp.int32)
            pltpu.sync_copy(
                tt_hbm.at[:, pl.ds(tail_start, 128)],
                bufs[0],
            )
            transpose(0, 64)
            pltpu.sync_copy(
                tbufs[0].at[pl.ds(0, 32)],
                out_hbm.at[pl.ds(n_full * 64, 32)],
            )

    return t(tt).reshape(v, d)


@functools.partial(jax.jit, static_argnums=(2, 3, 4))
def _gather_rows(indices, table, b, h, d):
    rows_per_w = b // NUM_WORKERS
    n_chunks = rows_per_w // CB
    n_groups = n_chunks // NBUF
    mesh = plsc.VectorSubcoreMesh(core_axis_name="c", subcore_axis_name="s")

    @functools.partial(
        pl.kernel,
        mesh=mesh,
        out_type=jax.ShapeDtypeStruct((b, h, d), jnp.float32),
        scratch_types=[
            pltpu.VMEM((rows_per_w, h), jnp.int32),
            [pltpu.VMEM((CB, h, d), jnp.float32) for _ in range(NBUF)],
            [pltpu.SemaphoreType.DMA for _ in range(NBUF)],
            [pltpu.SemaphoreType.DMA for _ in range(NBUF)],
        ],
        compiler_params=pltpu.CompilerParams(use_tc_tiling_on_sc=False),
    )
    def k(idx_hbm, table_hbm, out_hbm, idx_v, rows, semg, semw):
        wid = lax.axis_index("s") * 2 + lax.axis_index("c")
        base = wid * rows_per_w

        def start_gather(c, p):
            for k_ in range(CB):
                pltpu.async_copy(
                    table_hbm.at[idx_v.at[c * CB + k_]], rows[p].at[k_], semg[p]
                )

        def wait_gather(c, p):
            for k_ in range(CB):
                pltpu.make_async_copy(
                    table_hbm.at[idx_v.at[c * CB + k_]], rows[p].at[k_], semg[p]
                ).wait()

        def start_write(c, p):
            pltpu.async_copy(
                rows[p], out_hbm.at[pl.ds(base + c * CB, CB)], semw[p]
            )

        def wait_write(c, p):
            pltpu.make_async_copy(
                rows[p], out_hbm.at[pl.ds(base + c * CB, CB)], semw[p]
            ).wait()

        pltpu.sync_copy(idx_hbm.at[pl.ds(base, rows_per_w)], idx_v)

        for p in range(NBUF):
            start_gather(p, p)
            if p >= 2:
                wait_gather(p - 2, p - 2)
                start_write(p - 2, p - 2)

        def body(g, carry):
            for p in range(NBUF):
                c = g * NBUF + p
                wait_write(c - NBUF, p)
                start_gather(c, p)
                wait_gather(c - 2, (p - 2) % NBUF)
                start_write(c - 2, (p - 2) % NBUF)
            return carry

        lax.fori_loop(1, n_groups, body, 0)

        n = n_chunks
        for c in (n - 2, n - 1):
            p = c % NBUF
            wait_gather(c, p)
            start_write(c, p)
        for p in range(NBUF):
            wait_write(n - NBUF + p, p)

    return k(indices, table)


def kernel(indices, table):
    b, h = indices.shape
    v, d = table.shape
    table_lin = _repack_table(table, v, d)
    return _gather_rows(indices, table_lin, b, h, d)


# speedup vs baseline: 1.1515x; 1.1515x over previous
import functools

import jax
import jax.numpy as jnp
from jax import lax
from jax.experimental import pallas as pl
from jax.experimental.pallas import tpu as pltpu
from jax.experimental.pallas import tpu_sc as plsc

NUM_WORKERS = 32
CB = 8
NBUF = 4


@functools.partial(jax.jit, static_argnums=(2, 3, 4))
def _gather_rows(indices, table, b, h, d):
    rows_per_w = b // NUM_WORKERS
    n_chunks = rows_per_w // CB
    n_groups = n_chunks // NBUF
    mesh = plsc.VectorSubcoreMesh(core_axis_name="c", subcore_axis_name="s")

    @functools.partial(
        pl.kernel,
        mesh=mesh,
        out_type=jax.ShapeDtypeStruct((b, h, d), jnp.float32),
        scratch_types=[
            pltpu.VMEM((rows_per_w, h), jnp.int32),
            [pltpu.VMEM((CB, h, d), jnp.float32) for _ in range(NBUF)],
            [pltpu.SemaphoreType.DMA for _ in range(NBUF)],
            [pltpu.SemaphoreType.DMA for _ in range(NBUF)],
        ],
        compiler_params=pltpu.CompilerParams(use_tc_tiling_on_sc=False),
    )
    def k(idx_hbm, table_hbm, out_hbm, idx_v, rows, semg, semw):
        wid = lax.axis_index("s") * 2 + lax.axis_index("c")
        base = wid * rows_per_w

        def start_gather(c, p):
            for k_ in range(CB):
                pltpu.async_copy(
                    table_hbm.at[idx_v.at[c * CB + k_]], rows[p].at[k_], semg[p]
                )

        def wait_gather(c, p):
            for k_ in range(CB):
                pltpu.make_async_copy(
                    table_hbm.at[idx_v.at[c * CB + k_]], rows[p].at[k_], semg[p]
                ).wait()

        def start_write(c, p):
            pltpu.async_copy(
                rows[p], out_hbm.at[pl.ds(base + c * CB, CB)], semw[p]
            )

        def wait_write(c, p):
            pltpu.make_async_copy(
                rows[p], out_hbm.at[pl.ds(base + c * CB, CB)], semw[p]
            ).wait()

        pltpu.sync_copy(idx_hbm.at[pl.ds(base, rows_per_w)], idx_v)

        for p in range(NBUF):
            start_gather(p, p)
            if p >= 2:
                wait_gather(p - 2, p - 2)
                start_write(p - 2, p - 2)

        def body(g, carry):
            for p in range(NBUF):
                c = g * NBUF + p
                wait_write(c - NBUF, p)
                start_gather(c, p)
                wait_gather(c - 2, (p - 2) % NBUF)
                start_write(c - 2, (p - 2) % NBUF)
            return carry

        lax.fori_loop(1, n_groups, body, 0)

        n = n_chunks
        for c in (n - 2, n - 1):
            p = c % NBUF
            wait_gather(c, p)
            start_write(c, p)
        for p in range(NBUF):
            wait_write(n - NBUF + p, p)

    return k(indices, table)


def kernel(indices, table):
    b, h = indices.shape
    v, d = table.shape
    return _gather_rows(indices, table, b, h, d)
